# Initial kernel scaffold; baseline (speedup 1.0000x reference)
import jax
import jax.numpy as jnp
from jax.experimental import pallas as pl


def kernel(hidden_feats, solv_node_feats, edge_index, node_graph_ids, W_emb, b_emb, gcn_W, gcn_b, lin2_W, lin2_b, lin3_W1, lin3_b1, lin3_W2, lin3_b2):
    raise NotImplementedError("write your pallas kernel here")


# speedup vs baseline: 4.5272x; 4.5272x over previous
import functools

import jax
import jax.numpy as jnp
from jax import lax
from jax.experimental import pallas as pl
from jax.experimental.pallas import tpu as pltpu
from jax.experimental.pallas import tpu_sc as plsc

N = 50000
E = 800000
B = 256
F = 64
HID = 64
N_GCN = 4
NC = 2
NS = 16
HALF = N // NC
SLICE = 1568
ACC_ROWS = NS * SLICE
TRASH = HALF
CH = 128
NCHUNKS = E // CH
DEGW = 8
RB = 2000

_mesh = plsc.VectorSubcoreMesh(core_axis_name="c", subcore_axis_name="s")


def _tile_ids():
    c = lax.axis_index("c")
    s = lax.axis_index("s")
    return c, s


def _num_chunks_for(s):
    return jnp.where(s < (NCHUNKS % NS), NCHUNKS // NS + 1, NCHUNKS // NS)


def _zero_acc_slice(zbuf, acc, s, width):
    del width
    z0 = s * SLICE
    for k in range(SLICE // CH):
        pltpu.sync_copy(zbuf, acc.at[pl.ds(z0 + k * CH, CH)])
    pltpu.sync_copy(zbuf, acc.at[pl.ds(z0 + SLICE - CH, CH)])


def _writeback(acc, out_hbm, c, s):
    lo = jnp.minimum(s * SLICE, HALF - SLICE)
    pltpu.sync_copy(acc.at[pl.ds(lo, SLICE)],
                    out_hbm.at[pl.ds(c * HALF + lo, SLICE)])


def _compute_ldst(dstv, ldst, base):
    for k in range(CH // 16):
        d = dstv[pl.ds(k * 16, 16)]
        l = d - base
        ok = (l >= 0) & (l < HALF)
        ldst[0, pl.ds(k * 16, 16)] = jnp.where(ok, l, TRASH)


def _sc_deg_body(dst_hbm, ones_hbm, zeros_hbm, deg_hbm, dstv, ldst, onesv,
                 zbuf, acc):
    c, s = _tile_ids()
    base = c * HALF
    pltpu.sync_copy(ones_hbm, onesv)
    pltpu.sync_copy(zeros_hbm, zbuf)
    _zero_acc_slice(zbuf, acc, s, DEGW)
    plsc.subcore_barrier()

    def body(t, carry):
        off = (t * NS + s) * CH
        pltpu.sync_copy(dst_hbm.at[pl.ds(off, CH)], dstv)
        _compute_ldst(dstv, ldst, base)
        pltpu.sync_copy(onesv, acc.at[ldst.at[0]], add=True)
        return carry

    lax.fori_loop(0, _num_chunks_for(s), body, 0)
    plsc.subcore_barrier()
    _writeback(acc, deg_hbm, c, s)


def _sc_agg_body(g_hbm, src_hbm, dst_hbm, zeros_hbm, agg_hbm, srcv, dstv,
                 ldst, staged, zbuf, acc):
    c, s = _tile_ids()
    base = c * HALF
    pltpu.sync_copy(zeros_hbm, zbuf)
    _zero_acc_slice(zbuf, acc, s, F)
    plsc.subcore_barrier()

    def body(t, carry):
        off = (t * NS + s) * CH
        pltpu.sync_copy(src_hbm.at[pl.ds(off, CH)], srcv)
        pltpu.sync_copy(dst_hbm.at[pl.ds(off, CH)], dstv)
        pltpu.sync_copy(g_hbm.at[srcv], staged)
        _compute_ldst(dstv, ldst, base)
        pltpu.sync_copy(staged, acc.at[ldst.at[0]], add=True)
        return carry

    lax.fori_loop(0, _num_chunks_for(s), body, 0)
    plsc.subcore_barrier()
    _writeback(acc, agg_hbm, c, s)


_sc_params = pltpu.CompilerParams(use_tc_tiling_on_sc=False)

_sc_deg = pl.kernel(
    _sc_deg_body,
    out_type=jax.ShapeDtypeStruct((N, DEGW), jnp.float32),
    mesh=_mesh,
    compiler_params=_sc_params,
    scratch_types=[
        pltpu.VMEM((CH,), jnp.int32),
        pltpu.VMEM((1, CH), jnp.int32),
        pltpu.VMEM((CH, DEGW), jnp.float32),
        pltpu.VMEM((CH, DEGW), jnp.float32),
        pltpu.VMEM_SHARED((ACC_ROWS, DEGW), jnp.float32),
    ],
)

_sc_agg = pl.kernel(
    _sc_agg_body,
    out_type=jax.ShapeDtypeStruct((N, F), jnp.float32),
    mesh=_mesh,
    compiler_params=_sc_params,
    scratch_types=[
        pltpu.VMEM((CH,), jnp.int32),
        pltpu.VMEM((CH,), jnp.int32),
        pltpu.VMEM((1, CH), jnp.int32),
        pltpu.VMEM((CH, F), jnp.float32),
        pltpu.VMEM((CH, F), jnp.float32),
        pltpu.VMEM_SHARED((ACC_ROWS, F), jnp.float32),
    ],
)


def _tc_init_body(solv_ref, wemb_ref, bemb_ref, deg_ref, h_ref, g_ref,
                  dinv_ref):
    dinv = lax.rsqrt(jnp.maximum(deg_ref[:, 0:1], 1.0))
    h = jnp.dot(solv_ref[...], wemb_ref[...]) + bemb_ref[...]
    h_ref[...] = h
    g_ref[...] = h * dinv
    dinv_ref[...] = dinv


_tc_init = pl.pallas_call(
    _tc_init_body,
    grid=(N // RB,),
    in_specs=[
        pl.BlockSpec((RB, 128), lambda i: (i, 0)),
        pl.BlockSpec((128, F), lambda i: (0, 0)),
        pl.BlockSpec((1, F), lambda i: (0, 0)),
        pl.BlockSpec((RB, DEGW), lambda i: (i, 0)),
    ],
    out_specs=[
        pl.BlockSpec((RB, F), lambda i: (i, 0)),
        pl.BlockSpec((RB, F), lambda i: (i, 0)),
        pl.BlockSpec((RB, 1), lambda i: (i, 0)),
    ],
    out_shape=[
        jax.ShapeDtypeStruct((N, F), jnp.float32),
        jax.ShapeDtypeStruct((N, F), jnp.float32),
        jax.ShapeDtypeStruct((N, 1), jnp.float32),
    ],
)


def _tc_layer_body(agg_ref, h_ref, dinv_ref, w_ref, b_ref, hn_ref, gn_ref):
    dinv = dinv_ref[...]
    a = agg_ref[...] * dinv
    z = jnp.dot(a, w_ref[...]) + b_ref[...]
    hn = h_ref[...] + jnp.maximum(z, 0.0)
    hn_ref[...] = hn
    gn_ref[...] = hn * dinv


_tc_layer = pl.pallas_call(
    _tc_layer_body,
    grid=(N // RB,),
    in_specs=[
        pl.BlockSpec((RB, F), lambda i: (i, 0)),
        pl.BlockSpec((RB, F), lambda i: (i, 0)),
        pl.BlockSpec((RB, 1), lambda i: (i, 0)),
        pl.BlockSpec((F, F), lambda i: (0, 0)),
        pl.BlockSpec((1, F), lambda i: (0, 0)),
    ],
    out_specs=[
        pl.BlockSpec((RB, F), lambda i: (i, 0)),
        pl.BlockSpec((RB, F), lambda i: (i, 0)),
    ],
    out_shape=[
        jax.ShapeDtypeStruct((N, F), jnp.float32),
        jax.ShapeDtypeStruct((N, F), jnp.float32),
    ],
)


def _tc_final_body(ids_ref, h_ref, hidden_ref, l2w0_ref, l2b0_ref, l2w1_ref,
                   l2b1_ref, w1_ref, b1_ref, w2_ref, b2_ref, out_ref,
                   pooled_scr):
    i = pl.program_id(0)

    @pl.when(i == 0)
    def _():
        pooled_scr[...] = jnp.zeros_like(pooled_scr)

    onehot = (ids_ref[...] == lax.broadcasted_iota(jnp.int32, (1, B), 1)
              ).astype(jnp.float32)
    pooled_scr[...] += lax.dot_general(onehot, h_ref[...],
                                       (((0,), (0,)), ((), ())))

    @pl.when(i == pl.num_programs(0) - 1)
    def _():
        p = pooled_scr[...]
        p = jnp.maximum(jnp.dot(p, l2w0_ref[...]) + l2b0_ref[...], 0.0)
        p = jnp.maximum(jnp.dot(p, l2w1_ref[...]) + l2b1_ref[...], 0.0)
        hc = jnp.concatenate([hidden_ref[...], p], axis=1)
        hc = jnp.maximum(jnp.dot(hc, w1_ref[...]) + b1_ref[...], 0.0)
        out_ref[...] = jnp.dot(hc, w2_ref[...]) + b2_ref[...]


_tc_final = pl.pallas_call(
    _tc_final_body,
    grid=(N // RB,),
    in_specs=[
        pl.BlockSpec((RB, 1), lambda i: (i, 0)),
        pl.BlockSpec((RB, F), lambda i: (i, 0)),
        pl.BlockSpec((B, HID), lambda i: (0, 0)),
        pl.BlockSpec((F, F), lambda i: (0, 0)),
        pl.BlockSpec((1, F), lambda i: (0, 0)),
        pl.BlockSpec((F, F), lambda i: (0, 0)),
        pl.BlockSpec((1, F), lambda i: (0, 0)),
        pl.BlockSpec((HID + F, HID + F), lambda i: (0, 0)),
        pl.BlockSpec((1, HID + F), lambda i: (0, 0)),
        pl.BlockSpec((HID + F, F), lambda i: (0, 0)),
        pl.BlockSpec((1, F), lambda i: (0, 0)),
    ],
    out_specs=pl.BlockSpec((B, F), lambda i: (0, 0)),
    out_shape=jax.ShapeDtypeStruct((B, F), jnp.float32),
    scratch_shapes=[pltpu.VMEM((B, F), jnp.float32)],
)


def kernel(hidden_feats, solv_node_feats, edge_index, node_graph_ids, W_emb,
           b_emb, gcn_W, gcn_b, lin2_W, lin2_b, lin3_W1, lin3_b1, lin3_W2,
           lin3_b2):
    edge = edge_index.astype(jnp.int32)
    src = edge[0]
    dst = edge[1]
    ids = node_graph_ids.astype(jnp.int32).reshape(N, 1)
    ones8 = jnp.ones((CH, DEGW), jnp.float32)
    zeros8 = jnp.zeros((CH, DEGW), jnp.float32)
    zerosF = jnp.zeros((CH, F), jnp.float32)

    deg8 = _sc_deg(dst, ones8, zeros8)
    h, g, dinv = _tc_init(solv_node_feats, W_emb, b_emb.reshape(1, F), deg8)
    for i in range(N_GCN):
        agg = _sc_agg(g, src, dst, zerosF)
        h, g = _tc_layer(agg, h, dinv, gcn_W[i], gcn_b[i].reshape(1, F))
    out = _tc_final(ids, h, hidden_feats, lin2_W[0], lin2_b[0].reshape(1, F),
                    lin2_W[1], lin2_b[1].reshape(1, F), lin3_W1,
                    lin3_b1.reshape(1, HID + F), lin3_W2,
                    lin3_b2.reshape(1, F))
    return out


# speedup vs baseline: 6.7391x; 1.4886x over previous
import functools

import jax
import jax.numpy as jnp
from jax import lax
from jax.experimental import pallas as pl
from jax.experimental.pallas import tpu as pltpu
from jax.experimental.pallas import tpu_sc as plsc

N = 50000
E = 800000
B = 256
F = 64
HID = 64
N_GCN = 4
NC = 2
NS = 16
HALF = N // NC
SLICE = 1568
ACC_ROWS = NS * SLICE
TRASH = HALF
CH = 128
NCHUNKS = E // CH
DEGW = 8
RB = 2000

_mesh = plsc.VectorSubcoreMesh(core_axis_name="c", subcore_axis_name="s")


RD = 3
T0 = (NCHUNKS // NS) // RD * RD
NO = T0 // RD
NTAIL = NCHUNKS - T0 * NS


def _tile_ids():
    c = lax.axis_index("c")
    s = lax.axis_index("s")
    return c, s


def _zero_acc_slice(zeros_hbm, acc, s):
    pltpu.sync_copy(zeros_hbm, acc.at[pl.ds(s * SLICE, SLICE)])


def _writeback(acc, out_hbm, c, s):
    lo = jnp.minimum(s * SLICE, HALF - SLICE)
    pltpu.sync_copy(acc.at[pl.ds(lo, SLICE)],
                    out_hbm.at[pl.ds(c * HALF + lo, SLICE)])


def _compute_ldst(dstv, ldst, k, base):
    for q in range(CH // 16):
        d = dstv[k, pl.ds(q * 16, 16)]
        l = d - base
        ok = (l >= 0) & (l < HALF)
        ldst[k, pl.ds(q * 16, 16)] = jnp.where(ok, l, TRASH)


def _sc_deg_body(dst_hbm, ones_hbm, zeros_hbm, deg_hbm, dstv, ldst, onesv,
                 acc, semL0, semL1, semL2, semS0, semS1, semS2):
    c, s = _tile_ids()
    base = c * HALF
    semL = [semL0, semL1, semL2]
    semS = [semS0, semS1, semS2]
    pltpu.sync_copy(ones_hbm, onesv)
    _zero_acc_slice(zeros_hbm, acc, s)
    plsc.subcore_barrier()

    def load(t, k):
        off = (t * NS + s) * CH
        pltpu.async_copy(dst_hbm.at[pl.ds(off, CH)], dstv.at[k], semL[k])

    def wait_load(k):
        pltpu.make_async_copy(dst_hbm.at[pl.ds(0, CH)], dstv.at[k],
                              semL[k]).wait()

    def scat(k):
        pltpu.async_copy(onesv, acc.at[ldst.at[k]], semS[k], add=True)

    def wait_scat(k):
        pltpu.make_async_copy(onesv, acc.at[ldst.at[k]], semS[k]).wait()

    def stage2(k, base):
        wait_load(k)
        _compute_ldst(dstv, ldst, k, base)
        scat(k)

    def body(o, carry):
        for r in range(RD):
            t = o * RD + r

            @pl.when(t >= RD)
            def _():
                wait_scat(r)

            load(t, r)

            @pl.when(t >= 1)
            def _():
                stage2((r + RD - 1) % RD, base)

        return carry

    lax.fori_loop(0, NO, body, 0)
    stage2((T0 - 1) % RD, base)
    for r in range(RD):
        wait_scat(r)

    @pl.when(s < NTAIL)
    def _():
        off = (T0 * NS + s) * CH
        pltpu.sync_copy(dst_hbm.at[pl.ds(off, CH)], dstv.at[0])
        _compute_ldst(dstv, ldst, 0, base)
        pltpu.sync_copy(onesv, acc.at[ldst.at[0]], add=True)

    plsc.subcore_barrier()
    _writeback(acc, deg_hbm, c, s)


def _sc_agg_body(g_hbm, src_hbm, dst_hbm, zeros_hbm, agg_hbm, srcv, dstv,
                 ldst, staged, acc, semL0, semL1, semL2, semG0, semG1, semG2,
                 semS0, semS1, semS2):
    c, s = _tile_ids()
    base = c * HALF
    semL = [semL0, semL1, semL2]
    semG = [semG0, semG1, semG2]
    semS = [semS0, semS1, semS2]
    _zero_acc_slice(zeros_hbm, acc, s)
    plsc.subcore_barrier()

    def load(t, k):
        off = (t * NS + s) * CH
        pltpu.async_copy(src_hbm.at[pl.ds(off, CH)], srcv.at[k], semL[k])
        pltpu.async_copy(dst_hbm.at[pl.ds(off, CH)], dstv.at[k], semL[k])

    def wait_load(k):
        pltpu.make_async_copy(src_hbm.at[pl.ds(0, CH)], srcv.at[k],
                              semL[k]).wait()
        pltpu.make_async_copy(dst_hbm.at[pl.ds(0, CH)], dstv.at[k],
                              semL[k]).wait()

    def gath(k):
        pltpu.async_copy(g_hbm.at[srcv.at[k]], staged.at[k], semG[k])

    def wait_gath(k):
        pltpu.make_async_copy(g_hbm.at[srcv.at[k]], staged.at[k],
                              semG[k]).wait()

    def scat(k):
        pltpu.async_copy(staged.at[k], acc.at[ldst.at[k]], semS[k], add=True)

    def wait_scat(k):
        pltpu.make_async_copy(staged.at[k], acc.at[ldst.at[k]],
                              semS[k]).wait()

    def body(o, carry):
        for r in range(RD):
            t = o * RD + r

            @pl.when(t >= RD)
            def _():
                wait_scat(r)

            load(t, r)

            @pl.when(t >= 1)
            def _():
                k1 = (r + RD - 1) % RD
                wait_load(k1)
                gath(k1)
                _compute_ldst(dstv, ldst, k1, base)

            @pl.when(t >= 2)
            def _():
                k2 = (r + RD - 2) % RD
                wait_gath(k2)
                scat(k2)

        return carry

    lax.fori_loop(0, NO, body, 0)
    kA = (T0 - 1) % RD
    kB = (T0 - 2) % RD
    wait_load(kA)
    gath(kA)
    _compute_ldst(dstv, ldst, kA, base)
    wait_gath(kB)
    scat(kB)
    wait_gath(kA)
    scat(kA)
    for r in range(RD):
        wait_scat(r)

    @pl.when(s < NTAIL)
    def _():
        off = (T0 * NS + s) * CH
        pltpu.sync_copy(src_hbm.at[pl.ds(off, CH)], srcv.at[0])
        pltpu.sync_copy(dst_hbm.at[pl.ds(off, CH)], dstv.at[0])
        pltpu.sync_copy(g_hbm.at[srcv.at[0]], staged.at[0])
        _compute_ldst(dstv, ldst, 0, base)
        pltpu.sync_copy(staged.at[0], acc.at[ldst.at[0]], add=True)

    plsc.subcore_barrier()
    _writeback(acc, agg_hbm, c, s)


_sc_params = pltpu.CompilerParams(use_tc_tiling_on_sc=False)

_sc_deg = pl.kernel(
    _sc_deg_body,
    out_type=jax.ShapeDtypeStruct((N, DEGW), jnp.float32),
    mesh=_mesh,
    compiler_params=_sc_params,
    scratch_types=[
        pltpu.VMEM((RD, CH), jnp.int32),
        pltpu.VMEM((RD, CH), jnp.int32),
        pltpu.VMEM((CH, DEGW), jnp.float32),
        pltpu.VMEM_SHARED((ACC_ROWS, DEGW), jnp.float32),
    ] + [pltpu.SemaphoreType.DMA] * 6,
)

_sc_agg = pl.kernel(
    _sc_agg_body,
    out_type=jax.ShapeDtypeStruct((N, F), jnp.float32),
    mesh=_mesh,
    compiler_params=_sc_params,
    scratch_types=[
        pltpu.VMEM((RD, CH), jnp.int32),
        pltpu.VMEM((RD, CH), jnp.int32),
        pltpu.VMEM((RD, CH), jnp.int32),
        pltpu.VMEM((RD, CH, F), jnp.float32),
        pltpu.VMEM_SHARED((ACC_ROWS, F), jnp.float32),
    ] + [pltpu.SemaphoreType.DMA] * 9,
)


def _tc_init_body(solv_ref, wemb_ref, bemb_ref, deg_ref, h_ref, g_ref,
                  dinv_ref):
    dinv = lax.rsqrt(jnp.maximum(deg_ref[:, 0:1], 1.0))
    h = jnp.dot(solv_ref[...], wemb_ref[...]) + bemb_ref[...]
    h_ref[...] = h
    g_ref[...] = h * dinv
    dinv_ref[...] = dinv


_tc_init = pl.pallas_call(
    _tc_init_body,
    grid=(N // RB,),
    in_specs=[
        pl.BlockSpec((RB, 128), lambda i: (i, 0)),
        pl.BlockSpec((128, F), lambda i: (0, 0)),
        pl.BlockSpec((1, F), lambda i: (0, 0)),
        pl.BlockSpec((RB, DEGW), lambda i: (i, 0)),
    ],
    out_specs=[
        pl.BlockSpec((RB, F), lambda i: (i, 0)),
        pl.BlockSpec((RB, F), lambda i: (i, 0)),
        pl.BlockSpec((RB, 1), lambda i: (i, 0)),
    ],
    out_shape=[
        jax.ShapeDtypeStruct((N, F), jnp.float32),
        jax.ShapeDtypeStruct((N, F), jnp.float32),
        jax.ShapeDtypeStruct((N, 1), jnp.float32),
    ],
)


def _tc_layer_body(agg_ref, h_ref, dinv_ref, w_ref, b_ref, hn_ref, gn_ref):
    dinv = dinv_ref[...]
    a = agg_ref[...] * dinv
    z = jnp.dot(a, w_ref[...]) + b_ref[...]
    hn = h_ref[...] + jnp.maximum(z, 0.0)
    hn_ref[...] = hn
    gn_ref[...] = hn * dinv


_tc_layer = pl.pallas_call(
    _tc_layer_body,
    grid=(N // RB,),
    in_specs=[
        pl.BlockSpec((RB, F), lambda i: (i, 0)),
        pl.BlockSpec((RB, F), lambda i: (i, 0)),
        pl.BlockSpec((RB, 1), lambda i: (i, 0)),
        pl.BlockSpec((F, F), lambda i: (0, 0)),
        pl.BlockSpec((1, F), lambda i: (0, 0)),
    ],
    out_specs=[
        pl.BlockSpec((RB, F), lambda i: (i, 0)),
        pl.BlockSpec((RB, F), lambda i: (i, 0)),
    ],
    out_shape=[
        jax.ShapeDtypeStruct((N, F), jnp.float32),
        jax.ShapeDtypeStruct((N, F), jnp.float32),
    ],
)


def _tc_final_body(ids_ref, h_ref, hidden_ref, l2w0_ref, l2b0_ref, l2w1_ref,
                   l2b1_ref, w1_ref, b1_ref, w2_ref, b2_ref, out_ref,
                   pooled_scr):
    i = pl.program_id(0)

    @pl.when(i == 0)
    def _():
        pooled_scr[...] = jnp.zeros_like(pooled_scr)

    onehot = (ids_ref[...] == lax.broadcasted_iota(jnp.int32, (1, B), 1)
              ).astype(jnp.float32)
    pooled_scr[...] += lax.dot_general(onehot, h_ref[...],
                                       (((0,), (0,)), ((), ())))

    @pl.when(i == pl.num_programs(0) - 1)
    def _():
        p = pooled_scr[...]
        p = jnp.maximum(jnp.dot(p, l2w0_ref[...]) + l2b0_ref[...], 0.0)
        p = jnp.maximum(jnp.dot(p, l2w1_ref[...]) + l2b1_ref[...], 0.0)
        hc = jnp.concatenate([hidden_ref[...], p], axis=1)
        hc = jnp.maximum(jnp.dot(hc, w1_ref[...]) + b1_ref[...], 0.0)
        out_ref[...] = jnp.dot(hc, w2_ref[...]) + b2_ref[...]


_tc_final = pl.pallas_call(
    _tc_final_body,
    grid=(N // RB,),
    in_specs=[
        pl.BlockSpec((RB, 1), lambda i: (i, 0)),
        pl.BlockSpec((RB, F), lambda i: (i, 0)),
        pl.BlockSpec((B, HID), lambda i: (0, 0)),
        pl.BlockSpec((F, F), lambda i: (0, 0)),
        pl.BlockSpec((1, F), lambda i: (0, 0)),
        pl.BlockSpec((F, F), lambda i: (0, 0)),
        pl.BlockSpec((1, F), lambda i: (0, 0)),
        pl.BlockSpec((HID + F, HID + F), lambda i: (0, 0)),
        pl.BlockSpec((1, HID + F), lambda i: (0, 0)),
        pl.BlockSpec((HID + F, F), lambda i: (0, 0)),
        pl.BlockSpec((1, F), lambda i: (0, 0)),
    ],
    out_specs=pl.BlockSpec((B, F), lambda i: (0, 0)),
    out_shape=jax.ShapeDtypeStruct((B, F), jnp.float32),
    scratch_shapes=[pltpu.VMEM((B, F), jnp.float32)],
)


def kernel(hidden_feats, solv_node_feats, edge_index, node_graph_ids, W_emb,
           b_emb, gcn_W, gcn_b, lin2_W, lin2_b, lin3_W1, lin3_b1, lin3_W2,
           lin3_b2):
    edge = edge_index.astype(jnp.int32)
    src = edge[0]
    dst = edge[1]
    ids = node_graph_ids.astype(jnp.int32).reshape(N, 1)
    ones8 = jnp.ones((CH, DEGW), jnp.float32)
    zeros8 = jnp.zeros((SLICE, DEGW), jnp.float32)
    zerosF = jnp.zeros((SLICE, F), jnp.float32)

    deg8 = _sc_deg(dst, ones8, zeros8)
    h, g, dinv = _tc_init(solv_node_feats, W_emb, b_emb.reshape(1, F), deg8)
    for i in range(N_GCN):
        agg = _sc_agg(g, src, dst, zerosF)
        h, g = _tc_layer(agg, h, dinv, gcn_W[i], gcn_b[i].reshape(1, F))
    out = _tc_final(ids, h, hidden_feats, lin2_W[0], lin2_b[0].reshape(1, F),
                    lin2_W[1], lin2_b[1].reshape(1, F), lin3_W1,
                    lin3_b1.reshape(1, HID + F), lin3_W2,
                    lin3_b2.reshape(1, F))
    return out


# speedup vs baseline: 9.7963x; 1.4537x over previous
import functools

import jax
import jax.numpy as jnp
from jax import lax
from jax.experimental import pallas as pl
from jax.experimental.pallas import tpu as pltpu
from jax.experimental.pallas import tpu_sc as plsc

N = 50000
E = 800000
B = 256
F = 64
HID = 64
N_GCN = 4
NC = 2
NS = 16
HALF = N // NC
SLICE = 1568
ACC_ROWS = NS * SLICE
TRASH = HALF
CH = 128
NCHUNKS = E // CH
DEGW = 8
RB = 2000

_mesh = plsc.VectorSubcoreMesh(core_axis_name="c", subcore_axis_name="s")


RD = 3
T0 = NCHUNKS // NS
NTAIL = NCHUNKS - T0 * NS
FLUSH = 1024
CAPB = 1536
PCAP = 51200


def _tile_ids():
    c = lax.axis_index("c")
    s = lax.axis_index("s")
    return c, s


def _zero_acc_slice(zeros_hbm, acc, s):
    pltpu.sync_copy(zeros_hbm, acc.at[pl.ds(s * SLICE, SLICE)])


def _writeback(acc, out_hbm, c, s):
    lo = jnp.minimum(s * SLICE, HALF - SLICE)
    pltpu.sync_copy(acc.at[pl.ds(lo, SLICE)],
                    out_hbm.at[pl.ds(c * HALF + lo, SLICE)])


def _compute_ldst(dstv, ldst, k, base):
    for q in range(CH // 16):
        d = dstv[k, pl.ds(q * 16, 16)]
        l = d - base
        ok = (l >= 0) & (l < HALF)
        ldst[k, pl.ds(q * 16, 16)] = jnp.where(ok, l, TRASH)


def _sc_prep_body(src_hbm, dst_hbm, ones_hbm, zeros_hbm, deg_hbm, psrc_hbm,
                  pldst_hbm, cnt_hbm, srcv, dstv, ldst, onesv, obs, obl,
                  cntv, acc, semL0, semL1, semS0, semS1):
    c, s = _tile_ids()
    base = c * HALF
    semL = [semL0, semL1]
    semS = [semS0, semS1]
    pltpu.sync_copy(ones_hbm, onesv)
    _zero_acc_slice(zeros_hbm, acc, s)
    plsc.subcore_barrier()

    def load(t, k):
        off = (t * NS + s) * CH
        pltpu.async_copy(src_hbm.at[pl.ds(off, CH)], srcv.at[k], semL[k])
        pltpu.async_copy(dst_hbm.at[pl.ds(off, CH)], dstv.at[k], semL[k])

    def wait_load(k):
        pltpu.make_async_copy(src_hbm.at[pl.ds(0, CH)], srcv.at[k],
                              semL[k]).wait()
        pltpu.make_async_copy(dst_hbm.at[pl.ds(0, CH)], dstv.at[k],
                              semL[k]).wait()

    def scat(k):
        pltpu.async_copy(onesv, acc.at[ldst.at[k]], semS[k], add=True)

    def wait_scat(k):
        pltpu.make_async_copy(onesv, acc.at[ldst.at[k]], semS[k]).wait()

    def compact(k, p, wp):
        for q in range(CH // 16):
            sv = srcv[k, pl.ds(q * 16, 16)]
            lv = ldst[k, pl.ds(q * 16, 16)]
            m = lv < TRASH
            cs = plsc.cumsum(m.astype(jnp.int32))
            pos = p + cs - 1
            plsc.store_scatter(obs, [pos], sv, mask=m)
            plsc.store_scatter(obl, [pos], lv, mask=m)
            p = p + cs[15]
        do_flush = p >= FLUSH

        @pl.when(do_flush)
        def _():
            wpa = pl.multiple_of(wp, FLUSH)
            pltpu.sync_copy(obs.at[pl.ds(0, FLUSH)],
                            psrc_hbm.at[c, s, pl.ds(wpa, FLUSH)])
            pltpu.sync_copy(obl.at[pl.ds(0, FLUSH)],
                            pldst_hbm.at[c, s, pl.ds(wpa, FLUSH)])
            for gg in range((CAPB - FLUSH) // 16):
                obs[pl.ds(gg * 16, 16)] = obs[pl.ds(FLUSH + gg * 16, 16)]
                obl[pl.ds(gg * 16, 16)] = obl[pl.ds(FLUSH + gg * 16, 16)]

        p = jnp.where(do_flush, p - FLUSH, p)
        wp = jnp.where(do_flush, wp + FLUSH, wp)
        return p, wp

    load(0, 0)

    def body(o, carry):
        p, wp = carry
        for r in range(2):
            t = o * 2 + r

            @pl.when(t < T0 - 1)
            def _():
                load(t + 1, 1 - r)

            @pl.when(t >= 2)
            def _():
                wait_scat(r)

            wait_load(r)
            _compute_ldst(dstv, ldst, r, base)
            scat(r)
            p, wp = compact(r, p, wp)
        return p, wp

    p, wp = lax.fori_loop(0, T0 // 2, body, (jnp.int32(0), jnp.int32(0)))
    wait_scat(0)
    wait_scat(1)

    @pl.when(s < NTAIL)
    def _():
        off = (T0 * NS + s) * CH
        pltpu.sync_copy(src_hbm.at[pl.ds(off, CH)], srcv.at[0])
        pltpu.sync_copy(dst_hbm.at[pl.ds(off, CH)], dstv.at[0])

    @pl.when(s >= NTAIL)
    def _():
        for q in range(CH // 16):
            dstv[0, pl.ds(q * 16, 16)] = jnp.full((16,), N, jnp.int32)

    _compute_ldst(dstv, ldst, 0, base)

    @pl.when(s < NTAIL)
    def _():
        pltpu.sync_copy(onesv, acc.at[ldst.at[0]], add=True)

    p, wp = compact(0, p, wp)

    total = wp + p
    total_pad = ((jnp.maximum(total, 1) + (RD * CH - 1)) // (RD * CH)
                 ) * (RD * CH)
    ngrp = (total_pad - total + 15) // 16

    def padbody(i, carry2):
        pos = p + i * 16 + lax.broadcasted_iota(jnp.int32, (16,), 0)
        plsc.store_scatter(obs, [pos], jnp.zeros((16,), jnp.int32))
        plsc.store_scatter(obl, [pos], jnp.full((16,), TRASH, jnp.int32))
        return carry2

    lax.fori_loop(0, ngrp, padbody, 0)
    wpa = pl.multiple_of(wp, FLUSH)
    pltpu.sync_copy(obs, psrc_hbm.at[c, s, pl.ds(wpa, CAPB)])
    pltpu.sync_copy(obl, pldst_hbm.at[c, s, pl.ds(wpa, CAPB)])
    cntv[...] = jnp.full((16,), total_pad, jnp.int32)
    pltpu.sync_copy(cntv, cnt_hbm.at[c, s])
    plsc.subcore_barrier()
    _writeback(acc, deg_hbm, c, s)


def _sc_agg_body(g_hbm, psrc_hbm, pldst_hbm, cnt_hbm, zeros_hbm, agg_hbm,
                 srcv, ldst, staged, cntv, acc, semL0, semL1, semL2, semG0,
                 semG1, semG2, semS0, semS1, semS2):
    c, s = _tile_ids()
    semL = [semL0, semL1, semL2]
    semG = [semG0, semG1, semG2]
    semS = [semS0, semS1, semS2]
    pltpu.sync_copy(cnt_hbm.at[c, s], cntv)
    cnt = cntv[...][0]
    _zero_acc_slice(zeros_hbm, acc, s)
    plsc.subcore_barrier()

    def load(t, k):
        off = t * CH
        pltpu.async_copy(psrc_hbm.at[c, s, pl.ds(off, CH)], srcv.at[k],
                         semL[k])
        pltpu.async_copy(pldst_hbm.at[c, s, pl.ds(off, CH)], ldst.at[k],
                         semL[k])

    def wait_load(k):
        pltpu.make_async_copy(psrc_hbm.at[c, s, pl.ds(0, CH)], srcv.at[k],
                              semL[k]).wait()
        pltpu.make_async_copy(pldst_hbm.at[c, s, pl.ds(0, CH)], ldst.at[k],
                              semL[k]).wait()

    def gath(k):
        pltpu.async_copy(g_hbm.at[srcv.at[k]], staged.at[k], semG[k])

    def wait_gath(k):
        pltpu.make_async_copy(g_hbm.at[srcv.at[k]], staged.at[k],
                              semG[k]).wait()

    def scat(k):
        pltpu.async_copy(staged.at[k], acc.at[ldst.at[k]], semS[k], add=True)

    def wait_scat(k):
        pltpu.make_async_copy(staged.at[k], acc.at[ldst.at[k]],
                              semS[k]).wait()

    def body(o, carry):
        for r in range(RD):
            t = o * RD + r

            @pl.when(t >= RD)
            def _():
                wait_scat(r)

            load(t, r)

            @pl.when(t >= 1)
            def _():
                k1 = (r + RD - 1) % RD
                wait_load(k1)
                gath(k1)

            @pl.when(t >= 2)
            def _():
                k2 = (r + RD - 2) % RD
                wait_gath(k2)
                scat(k2)

        return carry

    lax.fori_loop(0, cnt // (RD * CH), body, 0)
    wait_load(RD - 1)
    gath(RD - 1)
    wait_gath(RD - 2)
    scat(RD - 2)
    wait_gath(RD - 1)
    scat(RD - 1)
    for r in range(RD):
        wait_scat(r)

    plsc.subcore_barrier()
    _writeback(acc, agg_hbm, c, s)


_sc_params = pltpu.CompilerParams(use_tc_tiling_on_sc=False,
                                  needs_layout_passes=False)

_sc_prep = pl.kernel(
    _sc_prep_body,
    out_type=(
        jax.ShapeDtypeStruct((N, DEGW), jnp.float32),
        jax.ShapeDtypeStruct((NC, NS, PCAP), jnp.int32),
        jax.ShapeDtypeStruct((NC, NS, PCAP), jnp.int32),
        jax.ShapeDtypeStruct((NC, NS, 16), jnp.int32),
    ),
    mesh=_mesh,
    compiler_params=_sc_params,
    scratch_types=[
        pltpu.VMEM((2, CH), jnp.int32),
        pltpu.VMEM((2, CH), jnp.int32),
        pltpu.VMEM((2, CH), jnp.int32),
        pltpu.VMEM((CH, DEGW), jnp.float32),
        pltpu.VMEM((CAPB,), jnp.int32),
        pltpu.VMEM((CAPB,), jnp.int32),
        pltpu.VMEM((16,), jnp.int32),
        pltpu.VMEM_SHARED((ACC_ROWS, DEGW), jnp.float32),
    ] + [pltpu.SemaphoreType.DMA] * 4,
)

_sc_agg = pl.kernel(
    _sc_agg_body,
    out_type=jax.ShapeDtypeStruct((N, F), jnp.float32),
    mesh=_mesh,
    compiler_params=_sc_params,
    scratch_types=[
        pltpu.VMEM((RD, CH), jnp.int32),
        pltpu.VMEM((RD, CH), jnp.int32),
        pltpu.VMEM((RD, CH, F), jnp.float32),
        pltpu.VMEM((16,), jnp.int32),
        pltpu.VMEM_SHARED((ACC_ROWS, F), jnp.float32),
    ] + [pltpu.SemaphoreType.DMA] * 9,
)


def _tc_init_body(solv_ref, wemb_ref, bemb_ref, deg_ref, h_ref, g_ref,
                  dinv_ref):
    dinv = lax.rsqrt(jnp.maximum(deg_ref[:, 0:1], 1.0))
    h = jnp.dot(solv_ref[...], wemb_ref[...]) + bemb_ref[...]
    h_ref[...] = h
    g_ref[...] = h * dinv
    dinv_ref[...] = dinv


_tc_init = pl.pallas_call(
    _tc_init_body,
    grid=(N // RB,),
    in_specs=[
        pl.BlockSpec((RB, 128), lambda i: (i, 0)),
        pl.BlockSpec((128, F), lambda i: (0, 0)),
        pl.BlockSpec((1, F), lambda i: (0, 0)),
        pl.BlockSpec((RB, DEGW), lambda i: (i, 0)),
    ],
    out_specs=[
        pl.BlockSpec((RB, F), lambda i: (i, 0)),
        pl.BlockSpec((RB, F), lambda i: (i, 0)),
        pl.BlockSpec((RB, 1), lambda i: (i, 0)),
    ],
    out_shape=[
        jax.ShapeDtypeStruct((N, F), jnp.float32),
        jax.ShapeDtypeStruct((N, F), jnp.float32),
        jax.ShapeDtypeStruct((N, 1), jnp.float32),
    ],
)


def _tc_layer_body(agg_ref, h_ref, dinv_ref, w_ref, b_ref, hn_ref, gn_ref):
    dinv = dinv_ref[...]
    a = agg_ref[...] * dinv
    z = jnp.dot(a, w_ref[...]) + b_ref[...]
    hn = h_ref[...] + jnp.maximum(z, 0.0)
    hn_ref[...] = hn
    gn_ref[...] = hn * dinv


_tc_layer = pl.pallas_call(
    _tc_layer_body,
    grid=(N // RB,),
    in_specs=[
        pl.BlockSpec((RB, F), lambda i: (i, 0)),
        pl.BlockSpec((RB, F), lambda i: (i, 0)),
        pl.BlockSpec((RB, 1), lambda i: (i, 0)),
        pl.BlockSpec((F, F), lambda i: (0, 0)),
        pl.BlockSpec((1, F), lambda i: (0, 0)),
    ],
    out_specs=[
        pl.BlockSpec((RB, F), lambda i: (i, 0)),
        pl.BlockSpec((RB, F), lambda i: (i, 0)),
    ],
    out_shape=[
        jax.ShapeDtypeStruct((N, F), jnp.float32),
        jax.ShapeDtypeStruct((N, F), jnp.float32),
    ],
)


def _tc_final_body(ids_ref, h_ref, hidden_ref, l2w0_ref, l2b0_ref, l2w1_ref,
                   l2b1_ref, w1_ref, b1_ref, w2_ref, b2_ref, out_ref,
                   pooled_scr):
    i = pl.program_id(0)

    @pl.when(i == 0)
    def _():
        pooled_scr[...] = jnp.zeros_like(pooled_scr)

    onehot = (ids_ref[...] == lax.broadcasted_iota(jnp.int32, (1, B), 1)
              ).astype(jnp.float32)
    pooled_scr[...] += lax.dot_general(onehot, h_ref[...],
                                       (((0,), (0,)), ((), ())))

    @pl.when(i == pl.num_programs(0) - 1)
    def _():
        p = pooled_scr[...]
        p = jnp.maximum(jnp.dot(p, l2w0_ref[...]) + l2b0_ref[...], 0.0)
        p = jnp.maximum(jnp.dot(p, l2w1_ref[...]) + l2b1_ref[...], 0.0)
        hc = jnp.concatenate([hidden_ref[...], p], axis=1)
        hc = jnp.maximum(jnp.dot(hc, w1_ref[...]) + b1_ref[...], 0.0)
        out_ref[...] = jnp.dot(hc, w2_ref[...]) + b2_ref[...]


_tc_final = pl.pallas_call(
    _tc_final_body,
    grid=(N // RB,),
    in_specs=[
        pl.BlockSpec((RB, 1), lambda i: (i, 0)),
        pl.BlockSpec((RB, F), lambda i: (i, 0)),
        pl.BlockSpec((B, HID), lambda i: (0, 0)),
        pl.BlockSpec((F, F), lambda i: (0, 0)),
        pl.BlockSpec((1, F), lambda i: (0, 0)),
        pl.BlockSpec((F, F), lambda i: (0, 0)),
        pl.BlockSpec((1, F), lambda i: (0, 0)),
        pl.BlockSpec((HID + F, HID + F), lambda i: (0, 0)),
        pl.BlockSpec((1, HID + F), lambda i: (0, 0)),
        pl.BlockSpec((HID + F, F), lambda i: (0, 0)),
        pl.BlockSpec((1, F), lambda i: (0, 0)),
    ],
    out_specs=pl.BlockSpec((B, F), lambda i: (0, 0)),
    out_shape=jax.ShapeDtypeStruct((B, F), jnp.float32),
    scratch_shapes=[pltpu.VMEM((B, F), jnp.float32)],
)


def kernel(hidden_feats, solv_node_feats, edge_index, node_graph_ids, W_emb,
           b_emb, gcn_W, gcn_b, lin2_W, lin2_b, lin3_W1, lin3_b1, lin3_W2,
           lin3_b2):
    edge = edge_index.astype(jnp.int32)
    src = edge[0]
    dst = edge[1]
    ids = node_graph_ids.astype(jnp.int32).reshape(N, 1)
    ones8 = jnp.ones((CH, DEGW), jnp.float32)
    zeros8 = jnp.zeros((SLICE, DEGW), jnp.float32)
    zerosF = jnp.zeros((SLICE, F), jnp.float32)

    deg8, psrc, pldst, cnt = _sc_prep(src, dst, ones8, zeros8)
    h, g, dinv = _tc_init(solv_node_feats, W_emb, b_emb.reshape(1, F), deg8)
    for i in range(N_GCN):
        agg = _sc_agg(g, psrc, pldst, cnt, zerosF)
        h, g = _tc_layer(agg, h, dinv, gcn_W[i], gcn_b[i].reshape(1, F))
    out = _tc_final(ids, h, hidden_feats, lin2_W[0], lin2_b[0].reshape(1, F),
                    lin2_W[1], lin2_b[1].reshape(1, F), lin3_W1,
                    lin3_b1.reshape(1, HID + F), lin3_W2,
                    lin3_b2.reshape(1, F))
    return out


# speedup vs baseline: 11.5102x; 1.1750x over previous
import functools

import jax
import jax.numpy as jnp
from jax import lax
from jax.experimental import pallas as pl
from jax.experimental.pallas import tpu as pltpu
from jax.experimental.pallas import tpu_sc as plsc

N = 50000
E = 800000
B = 256
F = 64
HID = 64
N_GCN = 4
NC = 2
NS = 16
HALF = N // NC
SLICE = 1568
ACC_ROWS = NS * SLICE
TRASH = HALF
CH = 128
NCHUNKS = E // CH
HISTP = 25024
RB = 2000

_mesh = plsc.VectorSubcoreMesh(core_axis_name="c", subcore_axis_name="s")


RD = 3
T0 = NCHUNKS // NS
NTAIL = NCHUNKS - T0 * NS
FLUSH = 1024
CAPB = 1536
PCAP = 51200


def _tile_ids():
    c = lax.axis_index("c")
    s = lax.axis_index("s")
    return c, s


def _zero_acc_slice(zeros_hbm, acc, s):
    pltpu.sync_copy(zeros_hbm, acc.at[pl.ds(s * SLICE, SLICE)])


def _writeback(acc, out_hbm, c, s):
    lo = jnp.minimum(s * SLICE, HALF - SLICE)
    pltpu.sync_copy(acc.at[pl.ds(lo, SLICE)],
                    out_hbm.at[pl.ds(c * HALF + lo, SLICE)])


def _compute_ldst(dstv, ldst, k, base):
    for q in range(CH // 16):
        d = dstv[k, pl.ds(q * 16, 16)]
        l = d - base
        ok = (l >= 0) & (l < HALF)
        ldst[k, pl.ds(q * 16, 16)] = jnp.where(ok, l, TRASH)


def _sc_prep_body(src_hbm, dst_hbm, deg_hbm, psrc_hbm,
                  pldst_hbm, cnt_hbm, srcv, dstv, ldst, obs, obl,
                  cntv, hist, rbuf, wbuf, acc2, semL0, semL1):
    c, s = _tile_ids()
    base = c * HALF
    semL = [semL0, semL1]
    fones = jnp.ones((16,), jnp.float32)

    def zbody(i, carry0):
        off = pl.multiple_of(i * 16, 16)
        hist[pl.ds(off, 16)] = jnp.zeros((16,), jnp.float32)
        return carry0

    lax.fori_loop(0, HISTP // 16, zbody, 0)

    def load(t, k):
        off = (t * NS + s) * CH
        pltpu.async_copy(src_hbm.at[pl.ds(off, CH)], srcv.at[k], semL[k])
        pltpu.async_copy(dst_hbm.at[pl.ds(off, CH)], dstv.at[k], semL[k])

    def wait_load(k):
        pltpu.make_async_copy(src_hbm.at[pl.ds(0, CH)], srcv.at[k],
                              semL[k]).wait()
        pltpu.make_async_copy(dst_hbm.at[pl.ds(0, CH)], dstv.at[k],
                              semL[k]).wait()

    def compact(k, p, wp):
        for q in range(CH // 16):
            sv = srcv[k, pl.ds(q * 16, 16)]
            lv = ldst[k, pl.ds(q * 16, 16)]
            m = lv < TRASH
            plsc.addupdate_scatter(hist, [lv], fones)
            cs = plsc.cumsum(m.astype(jnp.int32))
            pos = p + cs - 1
            plsc.store_scatter(obs, [pos], sv, mask=m)
            plsc.store_scatter(obl, [pos], lv, mask=m)
            p = p + cs[15]
        do_flush = p >= FLUSH

        @pl.when(do_flush)
        def _():
            wpa = pl.multiple_of(wp, FLUSH)
            pltpu.sync_copy(obs.at[pl.ds(0, FLUSH)],
                            psrc_hbm.at[c, s, pl.ds(wpa, FLUSH)])
            pltpu.sync_copy(obl.at[pl.ds(0, FLUSH)],
                            pldst_hbm.at[c, s, pl.ds(wpa, FLUSH)])
            for gg in range((CAPB - FLUSH) // 16):
                obs[pl.ds(gg * 16, 16)] = obs[pl.ds(FLUSH + gg * 16, 16)]
                obl[pl.ds(gg * 16, 16)] = obl[pl.ds(FLUSH + gg * 16, 16)]

        p = jnp.where(do_flush, p - FLUSH, p)
        wp = jnp.where(do_flush, wp + FLUSH, wp)
        return p, wp

    load(0, 0)

    def body(o, carry):
        p, wp = carry
        for r in range(2):
            t = o * 2 + r

            @pl.when(t < T0 - 1)
            def _():
                load(t + 1, 1 - r)

            wait_load(r)
            _compute_ldst(dstv, ldst, r, base)
            p, wp = compact(r, p, wp)
        return p, wp

    p, wp = lax.fori_loop(0, T0 // 2, body, (jnp.int32(0), jnp.int32(0)))

    @pl.when(s < NTAIL)
    def _():
        off = (T0 * NS + s) * CH
        pltpu.sync_copy(src_hbm.at[pl.ds(off, CH)], srcv.at[0])
        pltpu.sync_copy(dst_hbm.at[pl.ds(off, CH)], dstv.at[0])

    @pl.when(s >= NTAIL)
    def _():
        for q in range(CH // 16):
            dstv[0, pl.ds(q * 16, 16)] = jnp.full((16,), N, jnp.int32)

    _compute_ldst(dstv, ldst, 0, base)

    p, wp = compact(0, p, wp)

    total = wp + p
    total_pad = ((jnp.maximum(total, 1) + (RD * CH - 1)) // (RD * CH)
                 ) * (RD * CH)
    ngrp = (total_pad - total + 15) // 16

    def padbody(i, carry2):
        pos = p + i * 16 + lax.broadcasted_iota(jnp.int32, (16,), 0)
        plsc.store_scatter(obs, [pos], jnp.zeros((16,), jnp.int32))
        plsc.store_scatter(obl, [pos], jnp.full((16,), TRASH, jnp.int32))
        return carry2

    lax.fori_loop(0, ngrp, padbody, 0)
    wpa = pl.multiple_of(wp, FLUSH)
    pltpu.sync_copy(obs, psrc_hbm.at[c, s, pl.ds(wpa, CAPB)])
    pltpu.sync_copy(obl, pldst_hbm.at[c, s, pl.ds(wpa, CAPB)])
    cntv[...] = jnp.full((16,), total_pad, jnp.int32)
    pltpu.sync_copy(cntv, cnt_hbm.at[c, s])

    pltpu.sync_copy(hist, acc2.at[s])
    plsc.subcore_barrier()
    lo = pl.multiple_of(jnp.minimum(s * SLICE, HALF - SLICE), 8)
    for k in range(NS):
        pltpu.sync_copy(acc2.at[k, pl.ds(lo, SLICE)], rbuf.at[k])

    def redbody(j, carry3):
        off = pl.multiple_of(j * 16, 16)
        tot = rbuf[0, pl.ds(off, 16)]
        for k in range(1, NS):
            tot = tot + rbuf[k, pl.ds(off, 16)]
        wbuf[pl.ds(off, 16)] = tot
        return carry3

    lax.fori_loop(0, SLICE // 16, redbody, 0)
    pltpu.sync_copy(wbuf, deg_hbm.at[pl.ds(c * HALF + lo, SLICE)])


def _sc_agg_body(g_hbm, psrc_hbm, pldst_hbm, cnt_hbm, zeros_hbm, agg_hbm,
                 srcv, ldst, staged, cntv, acc, semL0, semL1, semL2, semG0,
                 semG1, semG2, semS0, semS1, semS2):
    c, s = _tile_ids()
    semL = [semL0, semL1, semL2]
    semG = [semG0, semG1, semG2]
    semS = [semS0, semS1, semS2]
    pltpu.sync_copy(cnt_hbm.at[c, s], cntv)
    cnt = cntv[...][0]
    _zero_acc_slice(zeros_hbm, acc, s)
    plsc.subcore_barrier()

    def load(t, k):
        off = t * CH
        pltpu.async_copy(psrc_hbm.at[c, s, pl.ds(off, CH)], srcv.at[k],
                         semL[k])
        pltpu.async_copy(pldst_hbm.at[c, s, pl.ds(off, CH)], ldst.at[k],
                         semL[k])

    def wait_load(k):
        pltpu.make_async_copy(psrc_hbm.at[c, s, pl.ds(0, CH)], srcv.at[k],
                              semL[k]).wait()
        pltpu.make_async_copy(pldst_hbm.at[c, s, pl.ds(0, CH)], ldst.at[k],
                              semL[k]).wait()

    def gath(k):
        pltpu.async_copy(g_hbm.at[srcv.at[k]], staged.at[k], semG[k])

    def wait_gath(k):
        pltpu.make_async_copy(g_hbm.at[srcv.at[k]], staged.at[k],
                              semG[k]).wait()

    def scat(k):
        pltpu.async_copy(staged.at[k], acc.at[ldst.at[k]], semS[k], add=True)

    def wait_scat(k):
        pltpu.make_async_copy(staged.at[k], acc.at[ldst.at[k]],
                              semS[k]).wait()

    def body(o, carry):
        for r in range(RD):
            t = o * RD + r

            @pl.when(t >= RD)
            def _():
                wait_scat(r)

            load(t, r)

            @pl.when(t >= 1)
            def _():
                k1 = (r + RD - 1) % RD
                wait_load(k1)
                gath(k1)

            @pl.when(t >= 2)
            def _():
                k2 = (r + RD - 2) % RD
                wait_gath(k2)
                scat(k2)

        return carry

    lax.fori_loop(0, cnt // (RD * CH), body, 0)
    wait_load(RD - 1)
    gath(RD - 1)
    wait_gath(RD - 2)
    scat(RD - 2)
    wait_gath(RD - 1)
    scat(RD - 1)
    for r in range(RD):
        wait_scat(r)

    plsc.subcore_barrier()
    _writeback(acc, agg_hbm, c, s)


_sc_params = pltpu.CompilerParams(use_tc_tiling_on_sc=False,
                                  needs_layout_passes=False)

_sc_prep = pl.kernel(
    _sc_prep_body,
    out_type=(
        jax.ShapeDtypeStruct((N,), jnp.float32),
        jax.ShapeDtypeStruct((NC, NS, PCAP), jnp.int32),
        jax.ShapeDtypeStruct((NC, NS, PCAP), jnp.int32),
        jax.ShapeDtypeStruct((NC, NS, 16), jnp.int32),
    ),
    mesh=_mesh,
    compiler_params=_sc_params,
    scratch_types=[
        pltpu.VMEM((2, CH), jnp.int32),
        pltpu.VMEM((2, CH), jnp.int32),
        pltpu.VMEM((2, CH), jnp.int32),
        pltpu.VMEM((CAPB,), jnp.int32),
        pltpu.VMEM((CAPB,), jnp.int32),
        pltpu.VMEM((16,), jnp.int32),
        pltpu.VMEM((HISTP,), jnp.float32),
        pltpu.VMEM((NS, SLICE), jnp.float32),
        pltpu.VMEM((SLICE,), jnp.float32),
        pltpu.VMEM_SHARED((NS, HISTP), jnp.float32),
    ] + [pltpu.SemaphoreType.DMA] * 2,
)

_sc_agg = pl.kernel(
    _sc_agg_body,
    out_type=jax.ShapeDtypeStruct((N, F), jnp.float32),
    mesh=_mesh,
    compiler_params=_sc_params,
    scratch_types=[
        pltpu.VMEM((RD, CH), jnp.int32),
        pltpu.VMEM((RD, CH), jnp.int32),
        pltpu.VMEM((RD, CH, F), jnp.float32),
        pltpu.VMEM((16,), jnp.int32),
        pltpu.VMEM_SHARED((ACC_ROWS, F), jnp.float32),
    ] + [pltpu.SemaphoreType.DMA] * 9,
)


def _tc_init_body(solv_ref, wemb_ref, bemb_ref, deg_ref, h_ref, g_ref,
                  dinv_ref):
    dinv = lax.rsqrt(jnp.maximum(deg_ref[...], 1.0))
    h = jnp.dot(solv_ref[...], wemb_ref[...]) + bemb_ref[...]
    h_ref[...] = h
    g_ref[...] = h * dinv
    dinv_ref[...] = dinv


_tc_init = pl.pallas_call(
    _tc_init_body,
    grid=(N // RB,),
    in_specs=[
        pl.BlockSpec((RB, 128), lambda i: (i, 0)),
        pl.BlockSpec((128, F), lambda i: (0, 0)),
        pl.BlockSpec((1, F), lambda i: (0, 0)),
        pl.BlockSpec((RB, 1), lambda i: (i, 0)),
    ],
    out_specs=[
        pl.BlockSpec((RB, F), lambda i: (i, 0)),
        pl.BlockSpec((RB, F), lambda i: (i, 0)),
        pl.BlockSpec((RB, 1), lambda i: (i, 0)),
    ],
    out_shape=[
        jax.ShapeDtypeStruct((N, F), jnp.float32),
        jax.ShapeDtypeStruct((N, F), jnp.float32),
        jax.ShapeDtypeStruct((N, 1), jnp.float32),
    ],
)


def _tc_layer_body(agg_ref, h_ref, dinv_ref, w_ref, b_ref, hn_ref, gn_ref):
    dinv = dinv_ref[...]
    a = agg_ref[...] * dinv
    z = jnp.dot(a, w_ref[...]) + b_ref[...]
    hn = h_ref[...] + jnp.maximum(z, 0.0)
    hn_ref[...] = hn
    gn_ref[...] = hn * dinv


_tc_layer = pl.pallas_call(
    _tc_layer_body,
    grid=(N // RB,),
    in_specs=[
        pl.BlockSpec((RB, F), lambda i: (i, 0)),
        pl.BlockSpec((RB, F), lambda i: (i, 0)),
        pl.BlockSpec((RB, 1), lambda i: (i, 0)),
        pl.BlockSpec((F, F), lambda i: (0, 0)),
        pl.BlockSpec((1, F), lambda i: (0, 0)),
    ],
    out_specs=[
        pl.BlockSpec((RB, F), lambda i: (i, 0)),
        pl.BlockSpec((RB, F), lambda i: (i, 0)),
    ],
    out_shape=[
        jax.ShapeDtypeStruct((N, F), jnp.float32),
        jax.ShapeDtypeStruct((N, F), jnp.float32),
    ],
)


def _tc_final_body(ids_ref, h_ref, hidden_ref, l2w0_ref, l2b0_ref, l2w1_ref,
                   l2b1_ref, w1_ref, b1_ref, w2_ref, b2_ref, out_ref,
                   pooled_scr):
    i = pl.program_id(0)

    @pl.when(i == 0)
    def _():
        pooled_scr[...] = jnp.zeros_like(pooled_scr)

    onehot = (ids_ref[...] == lax.broadcasted_iota(jnp.int32, (1, B), 1)
              ).astype(jnp.float32)
    pooled_scr[...] += lax.dot_general(onehot, h_ref[...],
                                       (((0,), (0,)), ((), ())))

    @pl.when(i == pl.num_programs(0) - 1)
    def _():
        p = pooled_scr[...]
        p = jnp.maximum(jnp.dot(p, l2w0_ref[...]) + l2b0_ref[...], 0.0)
        p = jnp.maximum(jnp.dot(p, l2w1_ref[...]) + l2b1_ref[...], 0.0)
        hc = jnp.concatenate([hidden_ref[...], p], axis=1)
        hc = jnp.maximum(jnp.dot(hc, w1_ref[...]) + b1_ref[...], 0.0)
        out_ref[...] = jnp.dot(hc, w2_ref[...]) + b2_ref[...]


_tc_final = pl.pallas_call(
    _tc_final_body,
    grid=(N // RB,),
    in_specs=[
        pl.BlockSpec((RB, 1), lambda i: (i, 0)),
        pl.BlockSpec((RB, F), lambda i: (i, 0)),
        pl.BlockSpec((B, HID), lambda i: (0, 0)),
        pl.BlockSpec((F, F), lambda i: (0, 0)),
        pl.BlockSpec((1, F), lambda i: (0, 0)),
        pl.BlockSpec((F, F), lambda i: (0, 0)),
        pl.BlockSpec((1, F), lambda i: (0, 0)),
        pl.BlockSpec((HID + F, HID + F), lambda i: (0, 0)),
        pl.BlockSpec((1, HID + F), lambda i: (0, 0)),
        pl.BlockSpec((HID + F, F), lambda i: (0, 0)),
        pl.BlockSpec((1, F), lambda i: (0, 0)),
    ],
    out_specs=pl.BlockSpec((B, F), lambda i: (0, 0)),
    out_shape=jax.ShapeDtypeStruct((B, F), jnp.float32),
    scratch_shapes=[pltpu.VMEM((B, F), jnp.float32)],
)


def kernel(hidden_feats, solv_node_feats, edge_index, node_graph_ids, W_emb,
           b_emb, gcn_W, gcn_b, lin2_W, lin2_b, lin3_W1, lin3_b1, lin3_W2,
           lin3_b2):
    edge = edge_index.astype(jnp.int32)
    src = edge[0]
    dst = edge[1]
    ids = node_graph_ids.astype(jnp.int32).reshape(N, 1)
    zerosF = jnp.zeros((SLICE, F), jnp.float32)

    deg, psrc, pldst, cnt = _sc_prep(src, dst)
    h, g, dinv = _tc_init(solv_node_feats, W_emb, b_emb.reshape(1, F),
                          deg.reshape(N, 1))
    for i in range(N_GCN):
        agg = _sc_agg(g, psrc, pldst, cnt, zerosF)
        h, g = _tc_layer(agg, h, dinv, gcn_W[i], gcn_b[i].reshape(1, F))
    out = _tc_final(ids, h, hidden_feats, lin2_W[0], lin2_b[0].reshape(1, F),
                    lin2_W[1], lin2_b[1].reshape(1, F), lin3_W1,
                    lin3_b1.reshape(1, HID + F), lin3_W2,
                    lin3_b2.reshape(1, F))
    return out
